# Initial kernel scaffold; baseline (speedup 1.0000x reference)
import jax
import jax.numpy as jnp
from jax.experimental import pallas as pl


def kernel(x, gso, W, b):
    raise NotImplementedError("write your pallas kernel here")


# speedup vs baseline: 1.0576x; 1.0576x over previous
import functools

import jax
import jax.numpy as jnp
from jax.experimental import pallas as pl


def _pass1_body(g_ref, x_ref, w1_ref, w2_ref, u_ref, v_ref):
    t1 = jnp.dot(g_ref[...], x_ref[...], preferred_element_type=jnp.float32)
    u_ref[...] = jnp.dot(t1, w2_ref[...], preferred_element_type=jnp.float32)
    v_ref[...] = jnp.dot(t1, w1_ref[...], preferred_element_type=jnp.float32)


def _pass2_body(g_ref, u_ref, x_ref, v_ref, wd_ref, b_ref, o_ref):
    acc = jnp.dot(g_ref[...], u_ref[...], preferred_element_type=jnp.float32)
    pre = (2.0 * acc
           + v_ref[...]
           + jnp.dot(x_ref[...], wd_ref[...], preferred_element_type=jnp.float32)
           + b_ref[...])
    m = jnp.max(pre, axis=1, keepdims=True)
    lse = jnp.log(jnp.sum(jnp.exp(pre - m), axis=1, keepdims=True)) + m
    o_ref[...] = pre - lse


def _pick_bm(n):
    for bm in (400, 200, 100, 80, 40, 16, 8):
        if n % bm == 0:
            return bm
    return n


@functools.partial(jax.jit, static_argnames=())
def kernel(x, gso, W, b):
    n, f = x.shape
    bm = _pick_bm(n)
    grid = (n // bm,)

    w0, w1, w2 = W[0], W[1], W[2]
    wd = (w0 - w2).astype(jnp.float32)
    b2 = b.reshape(1, f).astype(jnp.float32)

    row_blk = pl.BlockSpec((bm, n), lambda i: (i, 0))
    full_x = pl.BlockSpec((n, f), lambda i: (0, 0))
    full_w = pl.BlockSpec((f, f), lambda i: (0, 0))
    out_blk = pl.BlockSpec((bm, f), lambda i: (i, 0))

    u, v = pl.pallas_call(
        _pass1_body,
        grid=grid,
        in_specs=[row_blk, full_x, full_w, full_w],
        out_specs=[out_blk, out_blk],
        out_shape=[jax.ShapeDtypeStruct((n, f), jnp.float32)] * 2,
    )(gso, x, w1, w2)

    out = pl.pallas_call(
        _pass2_body,
        grid=grid,
        in_specs=[row_blk, full_x, out_blk, out_blk, full_w,
                  pl.BlockSpec((1, f), lambda i: (0, 0))],
        out_specs=out_blk,
        out_shape=jax.ShapeDtypeStruct((n, f), jnp.float32),
    )(gso, u, x, v, wd, b2)
    return out
